# Initial kernel scaffold; baseline (speedup 1.0000x reference)
import jax
import jax.numpy as jnp
from jax.experimental import pallas as pl


def kernel(x, edge_index, W1, b1, W2, b2):
    raise NotImplementedError("write your pallas kernel here")


# speedup vs baseline: 15.0849x; 15.0849x over previous
import jax
import jax.numpy as jnp
from jax import lax
from jax.experimental import pallas as pl
from jax.experimental.pallas import tpu as pltpu
from jax.experimental.pallas import tpu_sc as plsc

N = 10000
E = 320000
IN = 128
HID = 256
OUT = 128
FH = IN // 2

NC, NS = 2, 16
NW = NC * NS
CK = 128
NCHUNK = E // CK
BASE_CH = NCHUNK // NW
EXTRA = NCHUNK - BASE_CH * NW
RPT = 640
NP = NS * RPT
RPT_A = 628
N_ACC = NS * RPT_A

ROW_BLK = 512


def _mesh():
    return plsc.VectorSubcoreMesh(
        core_axis_name="c", subcore_axis_name="s",
        num_cores=NC, num_subcores=NS)


def _chunk_range(wid):
    start = wid * BASE_CH + jnp.minimum(wid, EXTRA)
    count = BASE_CH + jnp.where(wid < EXTRA, 1, 0)
    return start, count


def _sc_histogram(col2):
    def body(col_hbm, out_hbm, idx_v, ones_v, stage_v, acc_sh):
        c = lax.axis_index("c")
        s = lax.axis_index("s")
        wid = c * NS + s
        for j in range(CK // 16):
            ones_v[pl.ds(j * 16, 16)] = jnp.ones((16,), jnp.float32)
        for j in range(RPT // 16):
            stage_v[pl.ds(j * 16, 16)] = jnp.zeros((16,), jnp.float32)

        pltpu.sync_copy(stage_v, acc_sh.at[pl.ds(s * RPT, RPT)])
        plsc.subcore_barrier()
        start, count = _chunk_range(wid)

        @pl.loop(start, start + count)
        def _(g):
            pltpu.sync_copy(col_hbm.at[g], idx_v)
            pltpu.sync_copy(ones_v, acc_sh.at[idx_v], add=True)

        plsc.subcore_barrier()
        pltpu.sync_copy(acc_sh.at[pl.ds(s * RPT, RPT)], stage_v)
        pltpu.sync_copy(stage_v, out_hbm.at[c, pl.ds(s * RPT, RPT)])

    return pl.kernel(
        body,
        out_type=jax.ShapeDtypeStruct((NC, NP), jnp.float32),
        mesh=_mesh(),
        scratch_types=[
            pltpu.VMEM((CK,), jnp.int32),
            pltpu.VMEM((CK,), jnp.float32),
            pltpu.VMEM((RPT,), jnp.float32),
            pltpu.VMEM_SHARED((NP,), jnp.float32),
        ],
    )(col2)


def _sc_edge_aggregate(gidx2, sidx2, tblA, tblB):
    def body(g_hbm, s_hbm, tA_hbm, tB_hbm, outA_hbm, outB_hbm,
             gi_v, si_v, rows_v, stage_v, zero_v, acc_sh):
        c = lax.axis_index("c")
        s = lax.axis_index("s")
        wid = c * NS + s
        start, count = _chunk_range(wid)

        @pl.loop(0, RPT_A)
        def _(r):
            for j in range(FH // 16):
                zero_v[r, pl.ds(j * 16, 16)] = jnp.zeros((16,), jnp.float32)

        pltpu.sync_copy(zero_v, acc_sh.at[pl.ds(s * RPT_A, RPT_A), :])

        for half, (tbl_hbm, out_hbm) in enumerate(
                ((tA_hbm, outA_hbm), (tB_hbm, outB_hbm))):
            if half:
                pltpu.sync_copy(zero_v, acc_sh.at[pl.ds(s * RPT_A, RPT_A), :])
            plsc.subcore_barrier()

            @pl.loop(start, start + count)
            def _(g):
                pltpu.sync_copy(g_hbm.at[g], gi_v)
                pltpu.sync_copy(s_hbm.at[g], si_v)
                pltpu.sync_copy(tbl_hbm.at[gi_v], rows_v)
                pltpu.sync_copy(rows_v, acc_sh.at[si_v], add=True)

            plsc.subcore_barrier()
            pltpu.sync_copy(acc_sh.at[pl.ds(s * RPT_A, RPT_A), :], stage_v)
            pltpu.sync_copy(stage_v, out_hbm.at[c, pl.ds(s * RPT_A, RPT_A), :])

    out_t = jax.ShapeDtypeStruct((NC, NP, FH), jnp.float32)
    return pl.kernel(
        body,
        out_type=(out_t, out_t),
        mesh=_mesh(),
        compiler_params=pltpu.CompilerParams(use_tc_tiling_on_sc=False),
        scratch_types=[
            pltpu.VMEM((CK,), jnp.int32),
            pltpu.VMEM((CK,), jnp.int32),
            pltpu.VMEM((CK, FH), jnp.float32),
            pltpu.VMEM((RPT_A, FH), jnp.float32),
            pltpu.VMEM((RPT_A, FH), jnp.float32),
            pltpu.VMEM_SHARED((N_ACC, FH), jnp.float32),
        ],
    )(gidx2, sidx2, tblA, tblB)


def _tc_prep(degp_t, xp):
    def body(dp_ref, x_ref, yA_ref, yB_ref, dis_ref, dis2_ref):
        deg = dp_ref[:, 0:1] + dp_ref[:, 1:2]
        dis = jnp.where(deg > 0, lax.rsqrt(deg), 0.0)
        dis2 = lax.rsqrt(deg + 1.0)
        y = dis * x_ref[...]
        yA_ref[...] = y[:, :FH]
        yB_ref[...] = y[:, FH:]
        dis_ref[...] = dis
        dis2_ref[...] = dis2

    grid = (NP // ROW_BLK,)
    return pl.pallas_call(
        body,
        grid=grid,
        in_specs=[
            pl.BlockSpec((ROW_BLK, NC), lambda i: (i, 0)),
            pl.BlockSpec((ROW_BLK, IN), lambda i: (i, 0)),
        ],
        out_specs=[
            pl.BlockSpec((ROW_BLK, FH), lambda i: (i, 0)),
            pl.BlockSpec((ROW_BLK, FH), lambda i: (i, 0)),
            pl.BlockSpec((ROW_BLK, 1), lambda i: (i, 0)),
            pl.BlockSpec((ROW_BLK, 1), lambda i: (i, 0)),
        ],
        out_shape=[
            jax.ShapeDtypeStruct((NP, FH), jnp.float32),
            jax.ShapeDtypeStruct((NP, FH), jnp.float32),
            jax.ShapeDtypeStruct((NP, 1), jnp.float32),
            jax.ShapeDtypeStruct((NP, 1), jnp.float32),
        ],
    )(degp_t, xp)


def _tc_dense(zpA, zpB, dis, dis2, W1, b1r, W2):
    def body(zpA_ref, zpB_ref, dis_ref, dis2_ref, w1_ref, b1_ref, w2_ref,
             uA_ref, uB_ref):
        zb = jnp.concatenate([zpA_ref[0] + zpA_ref[1],
                              zpB_ref[0] + zpB_ref[1]], axis=1)
        a = dis_ref[...] * zb
        h = lax.dot_general(a, w1_ref[...], (((1,), (1,)), ((), ())),
                            preferred_element_type=jnp.float32) + b1_ref[...]
        xw = lax.dot_general(h, w2_ref[...], (((1,), (1,)), ((), ())),
                             preferred_element_type=jnp.float32)
        u = dis2_ref[...] * xw
        uA_ref[...] = u[:, :FH]
        uB_ref[...] = u[:, FH:]

    grid = (NP // ROW_BLK,)
    return pl.pallas_call(
        body,
        grid=grid,
        in_specs=[
            pl.BlockSpec((NC, ROW_BLK, FH), lambda i: (0, i, 0)),
            pl.BlockSpec((NC, ROW_BLK, FH), lambda i: (0, i, 0)),
            pl.BlockSpec((ROW_BLK, 1), lambda i: (i, 0)),
            pl.BlockSpec((ROW_BLK, 1), lambda i: (i, 0)),
            pl.BlockSpec((HID, IN), lambda i: (0, 0)),
            pl.BlockSpec((1, HID), lambda i: (0, 0)),
            pl.BlockSpec((OUT, HID), lambda i: (0, 0)),
        ],
        out_specs=[
            pl.BlockSpec((ROW_BLK, FH), lambda i: (i, 0)),
            pl.BlockSpec((ROW_BLK, FH), lambda i: (i, 0)),
        ],
        out_shape=[
            jax.ShapeDtypeStruct((NP, FH), jnp.float32),
            jax.ShapeDtypeStruct((NP, FH), jnp.float32),
        ],
    )(zpA, zpB, dis, dis2, W1, b1r, W2)


def _tc_final(vpA, vpB, uA, uB, dis2, b2r):
    def body(vpA_ref, vpB_ref, uA_ref, uB_ref, dis2_ref, b2_ref, out_ref):
        vb = jnp.concatenate(
            [vpA_ref[0] + vpA_ref[1] + uA_ref[...],
             vpB_ref[0] + vpB_ref[1] + uB_ref[...]], axis=1)
        out_ref[...] = dis2_ref[...] * vb + b2_ref[...]

    grid = (NP // ROW_BLK,)
    return pl.pallas_call(
        body,
        grid=grid,
        in_specs=[
            pl.BlockSpec((NC, ROW_BLK, FH), lambda i: (0, i, 0)),
            pl.BlockSpec((NC, ROW_BLK, FH), lambda i: (0, i, 0)),
            pl.BlockSpec((ROW_BLK, FH), lambda i: (i, 0)),
            pl.BlockSpec((ROW_BLK, FH), lambda i: (i, 0)),
            pl.BlockSpec((ROW_BLK, 1), lambda i: (i, 0)),
            pl.BlockSpec((1, OUT), lambda i: (0, 0)),
        ],
        out_specs=pl.BlockSpec((ROW_BLK, OUT), lambda i: (i, 0)),
        out_shape=jax.ShapeDtypeStruct((NP, OUT), jnp.float32),
    )(vpA, vpB, uA, uB, dis2, b2r)


def kernel(x, edge_index, W1, b1, W2, b2):
    row2 = edge_index[0].reshape(NCHUNK, CK)
    col2 = edge_index[1].reshape(NCHUNK, CK)
    xp = jnp.pad(x, ((0, NP - N), (0, 0)))

    degp = _sc_histogram(col2)
    yA, yB, dis, dis2 = _tc_prep(jnp.transpose(degp), xp)
    zpA, zpB = _sc_edge_aggregate(col2, row2, yA, yB)
    uA, uB = _tc_dense(zpA, zpB, dis, dis2, W1,
                       b1.reshape(1, HID), W2)
    vpA, vpB = _sc_edge_aggregate(row2, col2, uA, uB)
    out = _tc_final(vpA, vpB, uA, uB, dis2, b2.reshape(1, OUT))
    return out[:N]


# speedup vs baseline: 25.9502x; 1.7203x over previous
import jax
import jax.numpy as jnp
from jax import lax
from jax.experimental import pallas as pl
from jax.experimental.pallas import tpu as pltpu
from jax.experimental.pallas import tpu_sc as plsc

N = 10000
E = 320000
IN = 128
HID = 256
OUT = 128
FH = IN // 2

NC, NS = 2, 16
NW = NC * NS
CK = 128
NCHUNK = E // CK
BASE_CH = NCHUNK // NW
EXTRA = NCHUNK - BASE_CH * NW
RPT = 640
NP = NS * RPT
RPT_A = 628
N_ACC = NS * RPT_A

ROW_BLK = 512


def _mesh():
    return plsc.VectorSubcoreMesh(
        core_axis_name="c", subcore_axis_name="s",
        num_cores=NC, num_subcores=NS)


def _chunk_range(wid):
    start = wid * BASE_CH + jnp.minimum(wid, EXTRA)
    count = BASE_CH + jnp.where(wid < EXTRA, 1, 0)
    return start, count


def _sc_histogram(col2):
    def body(col_hbm, out_hbm, idx_v, ones_v, stage_v, acc_sh):
        c = lax.axis_index("c")
        s = lax.axis_index("s")
        wid = c * NS + s
        for j in range(CK // 16):
            ones_v[pl.ds(j * 16, 16)] = jnp.ones((16,), jnp.float32)
        for j in range(RPT // 16):
            stage_v[pl.ds(j * 16, 16)] = jnp.zeros((16,), jnp.float32)

        pltpu.sync_copy(stage_v, acc_sh.at[pl.ds(s * RPT, RPT)])
        plsc.subcore_barrier()
        start, count = _chunk_range(wid)

        @pl.loop(start, start + count)
        def _(g):
            pltpu.sync_copy(col_hbm.at[g], idx_v)
            pltpu.sync_copy(ones_v, acc_sh.at[idx_v], add=True)

        plsc.subcore_barrier()
        pltpu.sync_copy(acc_sh.at[pl.ds(s * RPT, RPT)], stage_v)
        pltpu.sync_copy(stage_v, out_hbm.at[c, pl.ds(s * RPT, RPT)])

    return pl.kernel(
        body,
        out_type=jax.ShapeDtypeStruct((NC, NP), jnp.float32),
        mesh=_mesh(),
        scratch_types=[
            pltpu.VMEM((CK,), jnp.int32),
            pltpu.VMEM((CK,), jnp.float32),
            pltpu.VMEM((RPT,), jnp.float32),
            pltpu.VMEM_SHARED((NP,), jnp.float32),
        ],
    )(col2)


def _sc_edge_aggregate(gidx2, sidx2, tblA, tblB):
    def body(g_hbm, s_hbm, tA_hbm, tB_hbm, outA_hbm, outB_hbm,
             gidx_v, sidx_v, gi_t, si_t, rows0_v, rows1_v, rowt_v,
             stage_v, acc_sh,
             sem_g0, sem_g1, sem_s0, sem_s1):
        c = lax.axis_index("c")
        s = lax.axis_index("s")
        wid = c * NS + s
        start, count = _chunk_range(wid)

        pltpu.sync_copy(g_hbm.at[pl.ds(start, BASE_CH)], gidx_v)
        pltpu.sync_copy(s_hbm.at[pl.ds(start, BASE_CH)], sidx_v)

        @pl.when(count > BASE_CH)
        def _():
            pltpu.sync_copy(g_hbm.at[start + BASE_CH], gi_t)
            pltpu.sync_copy(s_hbm.at[start + BASE_CH], si_t)

        for half, (tbl_hbm, out_hbm) in enumerate(
                ((tA_hbm, outA_hbm), (tB_hbm, outB_hbm))):
            @pl.loop(0, RPT_A // 4)
            def _(r):
                for rr in range(4):
                    for j in range(FH // 16):
                        stage_v[r * 4 + rr, pl.ds(j * 16, 16)] = (
                            jnp.zeros((16,), jnp.float32))

            pltpu.sync_copy(stage_v, acc_sh.at[pl.ds(s * RPT_A, RPT_A), :])
            plsc.subcore_barrier()

            pltpu.async_copy(tbl_hbm.at[gidx_v.at[0]], rows0_v, sem_g0)

            @pl.loop(0, BASE_CH, step=2)
            def _(k):
                pltpu.make_async_copy(
                    tbl_hbm.at[gidx_v.at[k]], rows0_v, sem_g0).wait()
                pltpu.async_copy(tbl_hbm.at[gidx_v.at[k + 1]], rows1_v, sem_g1)
                pltpu.async_copy(rows0_v, acc_sh.at[sidx_v.at[k]], sem_s0,
                                 add=True)
                pltpu.make_async_copy(
                    tbl_hbm.at[gidx_v.at[k + 1]], rows1_v, sem_g1).wait()
                pltpu.make_async_copy(
                    rows0_v, acc_sh.at[sidx_v.at[k]], sem_s0).wait()

                @pl.when(k + 2 < BASE_CH)
                def _():
                    pltpu.async_copy(tbl_hbm.at[gidx_v.at[k + 2]], rows0_v,
                                     sem_g0)

                pltpu.async_copy(rows1_v, acc_sh.at[sidx_v.at[k + 1]], sem_s1,
                                 add=True)
                pltpu.make_async_copy(
                    rows1_v, acc_sh.at[sidx_v.at[k + 1]], sem_s1).wait()

            @pl.when(count > BASE_CH)
            def _():
                pltpu.sync_copy(tbl_hbm.at[gi_t], rowt_v)
                pltpu.sync_copy(rowt_v, acc_sh.at[si_t], add=True)

            plsc.subcore_barrier()
            pltpu.sync_copy(acc_sh.at[pl.ds(s * RPT_A, RPT_A), :], stage_v)
            pltpu.sync_copy(stage_v, out_hbm.at[c, pl.ds(s * RPT_A, RPT_A), :])

    out_t = jax.ShapeDtypeStruct((NC, NP, FH), jnp.float32)
    return pl.kernel(
        body,
        out_type=(out_t, out_t),
        mesh=_mesh(),
        compiler_params=pltpu.CompilerParams(use_tc_tiling_on_sc=False),
        scratch_types=[
            pltpu.VMEM((BASE_CH, CK), jnp.int32),
            pltpu.VMEM((BASE_CH, CK), jnp.int32),
            pltpu.VMEM((CK,), jnp.int32),
            pltpu.VMEM((CK,), jnp.int32),
            pltpu.VMEM((CK, FH), jnp.float32),
            pltpu.VMEM((CK, FH), jnp.float32),
            pltpu.VMEM((CK, FH), jnp.float32),
            pltpu.VMEM((RPT_A, FH), jnp.float32),
            pltpu.VMEM_SHARED((N_ACC, FH), jnp.float32),
            pltpu.SemaphoreType.DMA,
            pltpu.SemaphoreType.DMA,
            pltpu.SemaphoreType.DMA,
            pltpu.SemaphoreType.DMA,
        ],
    )(gidx2, sidx2, tblA, tblB)


def _tc_prep(degp_t, xp):
    def body(dp_ref, x_ref, yA_ref, yB_ref, dis_ref, dis2_ref):
        deg = dp_ref[:, 0:1] + dp_ref[:, 1:2]
        dis = jnp.where(deg > 0, lax.rsqrt(deg), 0.0)
        dis2 = lax.rsqrt(deg + 1.0)
        y = dis * x_ref[...]
        yA_ref[...] = y[:, :FH]
        yB_ref[...] = y[:, FH:]
        dis_ref[...] = dis
        dis2_ref[...] = dis2

    grid = (NP // ROW_BLK,)
    return pl.pallas_call(
        body,
        grid=grid,
        in_specs=[
            pl.BlockSpec((ROW_BLK, NC), lambda i: (i, 0)),
            pl.BlockSpec((ROW_BLK, IN), lambda i: (i, 0)),
        ],
        out_specs=[
            pl.BlockSpec((ROW_BLK, FH), lambda i: (i, 0)),
            pl.BlockSpec((ROW_BLK, FH), lambda i: (i, 0)),
            pl.BlockSpec((ROW_BLK, 1), lambda i: (i, 0)),
            pl.BlockSpec((ROW_BLK, 1), lambda i: (i, 0)),
        ],
        out_shape=[
            jax.ShapeDtypeStruct((NP, FH), jnp.float32),
            jax.ShapeDtypeStruct((NP, FH), jnp.float32),
            jax.ShapeDtypeStruct((NP, 1), jnp.float32),
            jax.ShapeDtypeStruct((NP, 1), jnp.float32),
        ],
    )(degp_t, xp)


def _tc_dense(zpA, zpB, dis, dis2, W1, b1r, W2):
    def body(zpA_ref, zpB_ref, dis_ref, dis2_ref, w1_ref, b1_ref, w2_ref,
             uA_ref, uB_ref):
        zb = jnp.concatenate([zpA_ref[0] + zpA_ref[1],
                              zpB_ref[0] + zpB_ref[1]], axis=1)
        a = dis_ref[...] * zb
        h = lax.dot_general(a, w1_ref[...], (((1,), (1,)), ((), ())),
                            preferred_element_type=jnp.float32) + b1_ref[...]
        xw = lax.dot_general(h, w2_ref[...], (((1,), (1,)), ((), ())),
                             preferred_element_type=jnp.float32)
        u = dis2_ref[...] * xw
        uA_ref[...] = u[:, :FH]
        uB_ref[...] = u[:, FH:]

    grid = (NP // ROW_BLK,)
    return pl.pallas_call(
        body,
        grid=grid,
        in_specs=[
            pl.BlockSpec((NC, ROW_BLK, FH), lambda i: (0, i, 0)),
            pl.BlockSpec((NC, ROW_BLK, FH), lambda i: (0, i, 0)),
            pl.BlockSpec((ROW_BLK, 1), lambda i: (i, 0)),
            pl.BlockSpec((ROW_BLK, 1), lambda i: (i, 0)),
            pl.BlockSpec((HID, IN), lambda i: (0, 0)),
            pl.BlockSpec((1, HID), lambda i: (0, 0)),
            pl.BlockSpec((OUT, HID), lambda i: (0, 0)),
        ],
        out_specs=[
            pl.BlockSpec((ROW_BLK, FH), lambda i: (i, 0)),
            pl.BlockSpec((ROW_BLK, FH), lambda i: (i, 0)),
        ],
        out_shape=[
            jax.ShapeDtypeStruct((NP, FH), jnp.float32),
            jax.ShapeDtypeStruct((NP, FH), jnp.float32),
        ],
    )(zpA, zpB, dis, dis2, W1, b1r, W2)


def _tc_final(vpA, vpB, uA, uB, dis2, b2r):
    def body(vpA_ref, vpB_ref, uA_ref, uB_ref, dis2_ref, b2_ref, out_ref):
        vb = jnp.concatenate(
            [vpA_ref[0] + vpA_ref[1] + uA_ref[...],
             vpB_ref[0] + vpB_ref[1] + uB_ref[...]], axis=1)
        out_ref[...] = dis2_ref[...] * vb + b2_ref[...]

    grid = (NP // ROW_BLK,)
    return pl.pallas_call(
        body,
        grid=grid,
        in_specs=[
            pl.BlockSpec((NC, ROW_BLK, FH), lambda i: (0, i, 0)),
            pl.BlockSpec((NC, ROW_BLK, FH), lambda i: (0, i, 0)),
            pl.BlockSpec((ROW_BLK, FH), lambda i: (i, 0)),
            pl.BlockSpec((ROW_BLK, FH), lambda i: (i, 0)),
            pl.BlockSpec((ROW_BLK, 1), lambda i: (i, 0)),
            pl.BlockSpec((1, OUT), lambda i: (0, 0)),
        ],
        out_specs=pl.BlockSpec((ROW_BLK, OUT), lambda i: (i, 0)),
        out_shape=jax.ShapeDtypeStruct((NP, OUT), jnp.float32),
    )(vpA, vpB, uA, uB, dis2, b2r)


def kernel(x, edge_index, W1, b1, W2, b2):
    row2 = edge_index[0].reshape(NCHUNK, CK)
    col2 = edge_index[1].reshape(NCHUNK, CK)
    xp = jnp.pad(x, ((0, NP - N), (0, 0)))

    degp = _sc_histogram(col2)
    yA, yB, dis, dis2 = _tc_prep(jnp.transpose(degp), xp)
    zpA, zpB = _sc_edge_aggregate(col2, row2, yA, yB)
    uA, uB = _tc_dense(zpA, zpB, dis, dis2, W1,
                       b1.reshape(1, HID), W2)
    vpA, vpB = _sc_edge_aggregate(row2, col2, uA, uB)
    out = _tc_final(vpA, vpB, uA, uB, dis2, b2.reshape(1, OUT))
    return out[:N]


# speedup vs baseline: 35.4966x; 1.3679x over previous
import jax
import jax.numpy as jnp
from jax import lax
from jax.experimental import pallas as pl
from jax.experimental.pallas import tpu as pltpu
from jax.experimental.pallas import tpu_sc as plsc

N = 10000
E = 320000
IN = 128
HID = 256
OUT = 128

NC, NS = 2, 16
NW = NC * NS
CK = 128
CPW = 80
NCH = NW * CPW
E_PAD = NCH * CK
RPT = 640
NP = NS * RPT
RPT_A = 632
N_ACC = NS * RPT_A
NTRASH = 64

NCHUNK_H = E // CK
BASE_H = NCHUNK_H // NW
EXTRA_H = NCHUNK_H - BASE_H * NW

ROW_BLK = 512


def _mesh():
    return plsc.VectorSubcoreMesh(
        core_axis_name="c", subcore_axis_name="s",
        num_cores=NC, num_subcores=NS)


def _sc_histogram(col2):
    def body(col_hbm, out_hbm, idx_v, ones_v, stage_v, acc_sh):
        c = lax.axis_index("c")
        s = lax.axis_index("s")
        wid = c * NS + s
        for j in range(CK // 16):
            ones_v[pl.ds(j * 16, 16)] = jnp.ones((16,), jnp.float32)
        for j in range(RPT // 16):
            stage_v[pl.ds(j * 16, 16)] = jnp.zeros((16,), jnp.float32)

        pltpu.sync_copy(stage_v, acc_sh.at[pl.ds(s * RPT, RPT)])
        plsc.subcore_barrier()
        start = wid * BASE_H + jnp.minimum(wid, EXTRA_H)
        count = BASE_H + jnp.where(wid < EXTRA_H, 1, 0)

        @pl.loop(start, start + count)
        def _(g):
            pltpu.sync_copy(col_hbm.at[g], idx_v)
            pltpu.sync_copy(ones_v, acc_sh.at[idx_v], add=True)

        plsc.subcore_barrier()
        pltpu.sync_copy(acc_sh.at[pl.ds(s * RPT, RPT)], stage_v)
        pltpu.sync_copy(stage_v, out_hbm.at[c, pl.ds(s * RPT, RPT)])

    return pl.kernel(
        body,
        out_type=jax.ShapeDtypeStruct((NC, NP), jnp.float32),
        mesh=_mesh(),
        scratch_types=[
            pltpu.VMEM((CK,), jnp.int32),
            pltpu.VMEM((CK,), jnp.float32),
            pltpu.VMEM((RPT,), jnp.float32),
            pltpu.VMEM_SHARED((NP,), jnp.float32),
        ],
    )(col2)


def _sc_edge_aggregate(ei3, table):
    def body(ei_hbm, tbl_hbm, out_hbm,
             ib0, ib1, rows0, rows1, acc_sh,
             sem_i0, sem_i1, sem_g0, sem_g1, sem_s0, sem_s1, sem_w0, sem_w1):
        c = lax.axis_index("c")
        s = lax.axis_index("s")
        wid = c * NS + s
        base = wid * CPW

        @pl.loop(0, CK)
        def _(r):
            for j in range(IN // 16):
                rows0[r, pl.ds(j * 16, 16)] = jnp.zeros((16,), jnp.float32)

        for j in range(4):
            pltpu.sync_copy(rows0,
                            acc_sh.at[pl.ds(s * RPT_A + j * CK, CK), :])
        pltpu.sync_copy(rows0.at[pl.ds(0, RPT_A - 4 * CK), :],
                        acc_sh.at[pl.ds(s * RPT_A + 4 * CK, RPT_A - 4 * CK), :])
        plsc.subcore_barrier()

        pltpu.async_copy(ei_hbm.at[base], ib0, sem_i0)
        pltpu.make_async_copy(ei_hbm.at[base], ib0, sem_i0).wait()
        pltpu.async_copy(tbl_hbm.at[ib0.at[0]], rows0, sem_g0)
        pltpu.async_copy(ei_hbm.at[base + 1], ib1, sem_i1)

        @pl.loop(0, CPW, step=2)
        def _(k):
            cc = base + k
            pltpu.make_async_copy(ei_hbm.at[cc + 1], ib1, sem_i1).wait()
            pltpu.make_async_copy(tbl_hbm.at[ib0.at[0]], rows0, sem_g0).wait()
            pltpu.async_copy(tbl_hbm.at[ib1.at[0]], rows1, sem_g1)
            pltpu.async_copy(rows0, acc_sh.at[ib0.at[1]], sem_s0, add=True)
            pltpu.make_async_copy(rows0, acc_sh.at[ib0.at[1]], sem_s0).wait()

            @pl.when(k + 2 < CPW)
            def _():
                pltpu.async_copy(ei_hbm.at[cc + 2], ib0, sem_i0)

            pltpu.make_async_copy(tbl_hbm.at[ib1.at[0]], rows1, sem_g1).wait()
            pltpu.async_copy(rows1, acc_sh.at[ib1.at[1]], sem_s1, add=True)

            @pl.when(k + 2 < CPW)
            def _():
                pltpu.make_async_copy(ei_hbm.at[cc + 2], ib0, sem_i0).wait()
                pltpu.async_copy(tbl_hbm.at[ib0.at[0]], rows0, sem_g0)

            pltpu.make_async_copy(rows1, acc_sh.at[ib1.at[1]], sem_s1).wait()

            @pl.when(k + 2 < CPW)
            def _():
                pltpu.async_copy(ei_hbm.at[cc + 3], ib1, sem_i1)

        plsc.subcore_barrier()

        for j in range(5):
            nrows = CK if j < 4 else RPT_A - 4 * CK
            buf = rows0 if j % 2 == 0 else rows1
            sem = sem_w0 if j % 2 == 0 else sem_w1
            if j >= 2:
                pj = j - 2
                pnr = CK if pj < 4 else RPT_A - 4 * CK
                pbuf = rows0 if pj % 2 == 0 else rows1
                pltpu.make_async_copy(
                    pbuf.at[pl.ds(0, pnr), :],
                    out_hbm.at[c, pl.ds(s * RPT_A + pj * CK, pnr), :],
                    sem).wait()
            pltpu.sync_copy(acc_sh.at[pl.ds(s * RPT_A + j * CK, nrows), :],
                            buf.at[pl.ds(0, nrows), :])
            pltpu.async_copy(buf.at[pl.ds(0, nrows), :],
                             out_hbm.at[c, pl.ds(s * RPT_A + j * CK, nrows), :],
                             sem)
        for j in range(3, 5):
            nrows = CK if j < 4 else RPT_A - 4 * CK
            buf = rows0 if j % 2 == 0 else rows1
            sem = sem_w0 if j % 2 == 0 else sem_w1
            pltpu.make_async_copy(
                buf.at[pl.ds(0, nrows), :],
                out_hbm.at[c, pl.ds(s * RPT_A + j * CK, nrows), :],
                sem).wait()

    return pl.kernel(
        body,
        out_type=jax.ShapeDtypeStruct((NC, NP, IN), jnp.float32),
        mesh=_mesh(),
        scratch_types=[
            pltpu.VMEM((2, CK), jnp.int32),
            pltpu.VMEM((2, CK), jnp.int32),
            pltpu.VMEM((CK, IN), jnp.float32),
            pltpu.VMEM((CK, IN), jnp.float32),
            pltpu.VMEM_SHARED((N_ACC, IN), jnp.float32),
            pltpu.SemaphoreType.DMA,
            pltpu.SemaphoreType.DMA,
            pltpu.SemaphoreType.DMA,
            pltpu.SemaphoreType.DMA,
            pltpu.SemaphoreType.DMA,
            pltpu.SemaphoreType.DMA,
            pltpu.SemaphoreType.DMA,
            pltpu.SemaphoreType.DMA,
        ],
    )(ei3, table)


def _tc_prep(degp_t, xp):
    def body(dp_ref, x_ref, y_ref, dis_ref, dis2_ref):
        deg = dp_ref[:, 0:1] + dp_ref[:, 1:2]
        dis = jnp.where(deg > 0, lax.rsqrt(deg), 0.0)
        dis2 = lax.rsqrt(deg + 1.0)
        y_ref[...] = dis * x_ref[...]
        dis_ref[...] = dis
        dis2_ref[...] = dis2

    grid = (NP // ROW_BLK,)
    return pl.pallas_call(
        body,
        grid=grid,
        in_specs=[
            pl.BlockSpec((ROW_BLK, NC), lambda i: (i, 0)),
            pl.BlockSpec((ROW_BLK, IN), lambda i: (i, 0)),
        ],
        out_specs=[
            pl.BlockSpec((ROW_BLK, IN), lambda i: (i, 0)),
            pl.BlockSpec((ROW_BLK, 1), lambda i: (i, 0)),
            pl.BlockSpec((ROW_BLK, 1), lambda i: (i, 0)),
        ],
        out_shape=[
            jax.ShapeDtypeStruct((NP, IN), jnp.float32),
            jax.ShapeDtypeStruct((NP, 1), jnp.float32),
            jax.ShapeDtypeStruct((NP, 1), jnp.float32),
        ],
    )(degp_t, xp)


def _tc_dense(zp, dis, dis2, W1, b1r, W2):
    def body(zp_ref, dis_ref, dis2_ref, w1_ref, b1_ref, w2_ref, u_ref):
        zb = zp_ref[0] + zp_ref[1]
        a = dis_ref[...] * zb
        h = lax.dot_general(a, w1_ref[...], (((1,), (1,)), ((), ())),
                            preferred_element_type=jnp.float32) + b1_ref[...]
        xw = lax.dot_general(h, w2_ref[...], (((1,), (1,)), ((), ())),
                             preferred_element_type=jnp.float32)
        u_ref[...] = dis2_ref[...] * xw

    grid = (NP // ROW_BLK,)
    return pl.pallas_call(
        body,
        grid=grid,
        in_specs=[
            pl.BlockSpec((NC, ROW_BLK, IN), lambda i: (0, i, 0)),
            pl.BlockSpec((ROW_BLK, 1), lambda i: (i, 0)),
            pl.BlockSpec((ROW_BLK, 1), lambda i: (i, 0)),
            pl.BlockSpec((HID, IN), lambda i: (0, 0)),
            pl.BlockSpec((1, HID), lambda i: (0, 0)),
            pl.BlockSpec((OUT, HID), lambda i: (0, 0)),
        ],
        out_specs=pl.BlockSpec((ROW_BLK, OUT), lambda i: (i, 0)),
        out_shape=jax.ShapeDtypeStruct((NP, OUT), jnp.float32),
    )(zp, dis, dis2, W1, b1r, W2)


def _tc_final(vp, u, dis2, b2r):
    def body(vp_ref, u_ref, dis2_ref, b2_ref, out_ref):
        vb = vp_ref[0] + vp_ref[1] + u_ref[...]
        out_ref[...] = dis2_ref[...] * vb + b2_ref[...]

    grid = (NP // ROW_BLK,)
    return pl.pallas_call(
        body,
        grid=grid,
        in_specs=[
            pl.BlockSpec((NC, ROW_BLK, OUT), lambda i: (0, i, 0)),
            pl.BlockSpec((ROW_BLK, OUT), lambda i: (i, 0)),
            pl.BlockSpec((ROW_BLK, 1), lambda i: (i, 0)),
            pl.BlockSpec((1, OUT), lambda i: (0, 0)),
        ],
        out_specs=pl.BlockSpec((ROW_BLK, OUT), lambda i: (i, 0)),
        out_shape=jax.ShapeDtypeStruct((NP, OUT), jnp.float32),
    )(vp, u, dis2, b2r)


def _edge_chunks(gidx, sidx):
    pad = E_PAD - E
    lanes = jnp.arange(pad, dtype=jnp.int32) % NTRASH
    g_all = jnp.concatenate([gidx, lanes])
    s_all = jnp.concatenate([sidx, N_ACC - NTRASH + lanes])
    return jnp.stack([g_all.reshape(NCH, CK), s_all.reshape(NCH, CK)], axis=1)


def kernel(x, edge_index, W1, b1, W2, b2):
    row = edge_index[0]
    col = edge_index[1]
    col2 = col.reshape(NCHUNK_H, CK)
    xp = jnp.pad(x, ((0, NP - N), (0, 0)))
    ei_z = _edge_chunks(col, row)
    ei_v = _edge_chunks(row, col)

    degp = _sc_histogram(col2)
    y, dis, dis2 = _tc_prep(jnp.transpose(degp), xp)
    zp = _sc_edge_aggregate(ei_z, y)
    u = _tc_dense(zp, dis, dis2, W1, b1.reshape(1, HID), W2)
    vp = _sc_edge_aggregate(ei_v, u)
    out = _tc_final(vp, u, dis2, b2.reshape(1, OUT))
    return out[:N]


# speedup vs baseline: 35.9676x; 1.0133x over previous
import jax
import jax.numpy as jnp
from jax import lax
from jax.experimental import pallas as pl
from jax.experimental.pallas import tpu as pltpu
from jax.experimental.pallas import tpu_sc as plsc

N = 10000
E = 320000
IN = 128
HID = 256
OUT = 128

NC, NS = 2, 16
NW = NC * NS
CK = 128
CPW = 80
NCH = NW * CPW
E_PAD = NCH * CK
RPT = 640
NP = NS * RPT
RPT_A = 632
N_ACC = NS * RPT_A
NTRASH = 64

NCHUNK_H = E // CK
BASE_H = NCHUNK_H // NW
EXTRA_H = NCHUNK_H - BASE_H * NW

ROW_BLK = 512


def _mesh():
    return plsc.VectorSubcoreMesh(
        core_axis_name="c", subcore_axis_name="s",
        num_cores=NC, num_subcores=NS)


def _sc_histogram(ei3):
    def body(ei_hbm, out_hbm, ib0, ib1, ones_v, stage_v, acc_sh,
             sem_i0, sem_i1):
        c = lax.axis_index("c")
        s = lax.axis_index("s")
        wid = c * NS + s
        base = wid * CPW
        for j in range(CK // 16):
            ones_v[pl.ds(j * 16, 16)] = jnp.ones((16,), jnp.float32)
        for j in range(RPT // 16):
            stage_v[pl.ds(j * 16, 16)] = jnp.zeros((16,), jnp.float32)

        pltpu.sync_copy(stage_v, acc_sh.at[pl.ds(s * RPT, RPT)])
        plsc.subcore_barrier()

        pltpu.async_copy(ei_hbm.at[base, 1], ib0, sem_i0)

        @pl.loop(0, CPW, step=2)
        def _(k):
            cc = base + k
            pltpu.make_async_copy(ei_hbm.at[cc, 1], ib0, sem_i0).wait()
            pltpu.async_copy(ei_hbm.at[cc + 1, 1], ib1, sem_i1)
            pltpu.sync_copy(ones_v, acc_sh.at[ib0], add=True)
            pltpu.make_async_copy(ei_hbm.at[cc + 1, 1], ib1, sem_i1).wait()

            @pl.when(k + 2 < CPW)
            def _():
                pltpu.async_copy(ei_hbm.at[cc + 2, 1], ib0, sem_i0)

            pltpu.sync_copy(ones_v, acc_sh.at[ib1], add=True)

        plsc.subcore_barrier()
        pltpu.sync_copy(acc_sh.at[pl.ds(s * RPT, RPT)], stage_v)
        pltpu.sync_copy(stage_v, out_hbm.at[c, pl.ds(s * RPT, RPT)])

    return pl.kernel(
        body,
        out_type=jax.ShapeDtypeStruct((NC, NP), jnp.float32),
        mesh=_mesh(),
        scratch_types=[
            pltpu.VMEM((CK,), jnp.int32),
            pltpu.VMEM((CK,), jnp.int32),
            pltpu.VMEM((CK,), jnp.float32),
            pltpu.VMEM((RPT,), jnp.float32),
            pltpu.VMEM_SHARED((NP,), jnp.float32),
            pltpu.SemaphoreType.DMA,
            pltpu.SemaphoreType.DMA,
        ],
    )(ei3)


def _sc_edge_aggregate(ei3, table):
    def body(ei_hbm, tbl_hbm, out_hbm,
             ib0, ib1, rows0, rows1, acc_sh,
             sem_i0, sem_i1, sem_g0, sem_g1, sem_s0, sem_s1, sem_w0, sem_w1):
        c = lax.axis_index("c")
        s = lax.axis_index("s")
        wid = c * NS + s
        base = wid * CPW

        @pl.loop(0, CK)
        def _(r):
            for j in range(IN // 16):
                rows0[r, pl.ds(j * 16, 16)] = jnp.zeros((16,), jnp.float32)

        for j in range(4):
            pltpu.sync_copy(rows0,
                            acc_sh.at[pl.ds(s * RPT_A + j * CK, CK), :])
        pltpu.sync_copy(rows0.at[pl.ds(0, RPT_A - 4 * CK), :],
                        acc_sh.at[pl.ds(s * RPT_A + 4 * CK, RPT_A - 4 * CK), :])
        plsc.subcore_barrier()

        pltpu.async_copy(ei_hbm.at[base], ib0, sem_i0)
        pltpu.make_async_copy(ei_hbm.at[base], ib0, sem_i0).wait()
        pltpu.async_copy(tbl_hbm.at[ib0.at[0]], rows0, sem_g0)
        pltpu.async_copy(ei_hbm.at[base + 1], ib1, sem_i1)

        @pl.loop(0, CPW, step=2)
        def _(k):
            cc = base + k
            pltpu.make_async_copy(ei_hbm.at[cc + 1], ib1, sem_i1).wait()
            pltpu.make_async_copy(tbl_hbm.at[ib0.at[0]], rows0, sem_g0).wait()
            pltpu.async_copy(tbl_hbm.at[ib1.at[0]], rows1, sem_g1)
            pltpu.async_copy(rows0, acc_sh.at[ib0.at[1]], sem_s0, add=True)
            pltpu.make_async_copy(rows0, acc_sh.at[ib0.at[1]], sem_s0).wait()

            @pl.when(k + 2 < CPW)
            def _():
                pltpu.async_copy(ei_hbm.at[cc + 2], ib0, sem_i0)

            pltpu.make_async_copy(tbl_hbm.at[ib1.at[0]], rows1, sem_g1).wait()
            pltpu.async_copy(rows1, acc_sh.at[ib1.at[1]], sem_s1, add=True)

            @pl.when(k + 2 < CPW)
            def _():
                pltpu.make_async_copy(ei_hbm.at[cc + 2], ib0, sem_i0).wait()
                pltpu.async_copy(tbl_hbm.at[ib0.at[0]], rows0, sem_g0)

            pltpu.make_async_copy(rows1, acc_sh.at[ib1.at[1]], sem_s1).wait()

            @pl.when(k + 2 < CPW)
            def _():
                pltpu.async_copy(ei_hbm.at[cc + 3], ib1, sem_i1)

        plsc.subcore_barrier()

        for j in range(5):
            nrows = CK if j < 4 else RPT_A - 4 * CK
            buf = rows0 if j % 2 == 0 else rows1
            sem = sem_w0 if j % 2 == 0 else sem_w1
            if j >= 2:
                pj = j - 2
                pnr = CK if pj < 4 else RPT_A - 4 * CK
                pbuf = rows0 if pj % 2 == 0 else rows1
                pltpu.make_async_copy(
                    pbuf.at[pl.ds(0, pnr), :],
                    out_hbm.at[c, pl.ds(s * RPT_A + pj * CK, pnr), :],
                    sem).wait()
            pltpu.sync_copy(acc_sh.at[pl.ds(s * RPT_A + j * CK, nrows), :],
                            buf.at[pl.ds(0, nrows), :])
            pltpu.async_copy(buf.at[pl.ds(0, nrows), :],
                             out_hbm.at[c, pl.ds(s * RPT_A + j * CK, nrows), :],
                             sem)
        for j in range(3, 5):
            nrows = CK if j < 4 else RPT_A - 4 * CK
            buf = rows0 if j % 2 == 0 else rows1
            sem = sem_w0 if j % 2 == 0 else sem_w1
            pltpu.make_async_copy(
                buf.at[pl.ds(0, nrows), :],
                out_hbm.at[c, pl.ds(s * RPT_A + j * CK, nrows), :],
                sem).wait()

    return pl.kernel(
        body,
        out_type=jax.ShapeDtypeStruct((NC, NP, IN), jnp.float32),
        mesh=_mesh(),
        scratch_types=[
            pltpu.VMEM((2, CK), jnp.int32),
            pltpu.VMEM((2, CK), jnp.int32),
            pltpu.VMEM((CK, IN), jnp.float32),
            pltpu.VMEM((CK, IN), jnp.float32),
            pltpu.VMEM_SHARED((N_ACC, IN), jnp.float32),
            pltpu.SemaphoreType.DMA,
            pltpu.SemaphoreType.DMA,
            pltpu.SemaphoreType.DMA,
            pltpu.SemaphoreType.DMA,
            pltpu.SemaphoreType.DMA,
            pltpu.SemaphoreType.DMA,
            pltpu.SemaphoreType.DMA,
            pltpu.SemaphoreType.DMA,
        ],
    )(ei3, table)


def _tc_prep(degp_t, xp):
    def body(dp_ref, x_ref, y_ref, dis_ref, dis2_ref):
        deg = dp_ref[:, 0:1] + dp_ref[:, 1:2]
        dis = jnp.where(deg > 0, lax.rsqrt(deg), 0.0)
        dis2 = lax.rsqrt(deg + 1.0)
        y_ref[...] = dis * x_ref[...]
        dis_ref[...] = dis
        dis2_ref[...] = dis2

    grid = (NP // ROW_BLK,)
    return pl.pallas_call(
        body,
        grid=grid,
        in_specs=[
            pl.BlockSpec((ROW_BLK, NC), lambda i: (i, 0)),
            pl.BlockSpec((ROW_BLK, IN), lambda i: (i, 0)),
        ],
        out_specs=[
            pl.BlockSpec((ROW_BLK, IN), lambda i: (i, 0)),
            pl.BlockSpec((ROW_BLK, 1), lambda i: (i, 0)),
            pl.BlockSpec((ROW_BLK, 1), lambda i: (i, 0)),
        ],
        out_shape=[
            jax.ShapeDtypeStruct((NP, IN), jnp.float32),
            jax.ShapeDtypeStruct((NP, 1), jnp.float32),
            jax.ShapeDtypeStruct((NP, 1), jnp.float32),
        ],
    )(degp_t, xp)


def _tc_dense(zp, dis, dis2, W1, b1r, W2):
    def body(zp_ref, dis_ref, dis2_ref, w1_ref, b1_ref, w2_ref, u_ref):
        zb = zp_ref[0] + zp_ref[1]
        a = dis_ref[...] * zb
        h = lax.dot_general(a, w1_ref[...], (((1,), (1,)), ((), ())),
                            preferred_element_type=jnp.float32) + b1_ref[...]
        xw = lax.dot_general(h, w2_ref[...], (((1,), (1,)), ((), ())),
                             preferred_element_type=jnp.float32)
        u_ref[...] = dis2_ref[...] * xw

    grid = (NP // ROW_BLK,)
    return pl.pallas_call(
        body,
        grid=grid,
        in_specs=[
            pl.BlockSpec((NC, ROW_BLK, IN), lambda i: (0, i, 0)),
            pl.BlockSpec((ROW_BLK, 1), lambda i: (i, 0)),
            pl.BlockSpec((ROW_BLK, 1), lambda i: (i, 0)),
            pl.BlockSpec((HID, IN), lambda i: (0, 0)),
            pl.BlockSpec((1, HID), lambda i: (0, 0)),
            pl.BlockSpec((OUT, HID), lambda i: (0, 0)),
        ],
        out_specs=pl.BlockSpec((ROW_BLK, OUT), lambda i: (i, 0)),
        out_shape=jax.ShapeDtypeStruct((NP, OUT), jnp.float32),
    )(zp, dis, dis2, W1, b1r, W2)


def _tc_final(vp, u, dis2, b2r):
    def body(vp_ref, u_ref, dis2_ref, b2_ref, out_ref):
        vb = vp_ref[0] + vp_ref[1] + u_ref[...]
        out_ref[...] = dis2_ref[...] * vb + b2_ref[...]

    grid = (NP // ROW_BLK,)
    return pl.pallas_call(
        body,
        grid=grid,
        in_specs=[
            pl.BlockSpec((NC, ROW_BLK, OUT), lambda i: (0, i, 0)),
            pl.BlockSpec((ROW_BLK, OUT), lambda i: (i, 0)),
            pl.BlockSpec((ROW_BLK, 1), lambda i: (i, 0)),
            pl.BlockSpec((1, OUT), lambda i: (0, 0)),
        ],
        out_specs=pl.BlockSpec((ROW_BLK, OUT), lambda i: (i, 0)),
        out_shape=jax.ShapeDtypeStruct((NP, OUT), jnp.float32),
    )(vp, u, dis2, b2r)


def _edge_chunks(gidx, sidx):
    pad = E_PAD - E
    lanes = jnp.arange(pad, dtype=jnp.int32) % NTRASH
    g_all = jnp.concatenate([gidx, lanes])
    s_all = jnp.concatenate([sidx, N_ACC - NTRASH + lanes])
    return jnp.stack([g_all.reshape(NCH, CK), s_all.reshape(NCH, CK)], axis=1)


def kernel(x, edge_index, W1, b1, W2, b2):
    row = edge_index[0]
    col = edge_index[1]
    xp = jnp.pad(x, ((0, NP - N), (0, 0)))
    ei_z = _edge_chunks(col, row)
    ei_v = _edge_chunks(row, col)

    degp = _sc_histogram(ei_v)
    y, dis, dis2 = _tc_prep(jnp.transpose(degp), xp)
    zp = _sc_edge_aggregate(ei_z, y)
    u = _tc_dense(zp, dis, dis2, W1, b1.reshape(1, HID), W2)
    vp = _sc_edge_aggregate(ei_v, u)
    out = _tc_final(vp, u, dis2, b2.reshape(1, OUT))
    return out[:N]


# speedup vs baseline: 37.3323x; 1.0379x over previous
import jax
import jax.numpy as jnp
from jax import lax
from jax.experimental import pallas as pl
from jax.experimental.pallas import tpu as pltpu
from jax.experimental.pallas import tpu_sc as plsc

N = 10000
E = 320000
IN = 128
HID = 256
OUT = 128

NC, NS = 2, 16
NW = NC * NS
CK = 128
CPW = 80
NCH = NW * CPW
E_PAD = NCH * CK
RPT = 640
NP = NS * RPT
RPT_A = 632
N_ACC = NS * RPT_A
NTRASH = 64

NCHUNK_H = E // CK
BASE_H = NCHUNK_H // NW
EXTRA_H = NCHUNK_H - BASE_H * NW

ROW_BLK = 512

T_AGG = jnp.bfloat16
VW_AGG = 32


def _mesh():
    return plsc.VectorSubcoreMesh(
        core_axis_name="c", subcore_axis_name="s",
        num_cores=NC, num_subcores=NS)


def _sc_histogram(ei3):
    def body(ei_hbm, out_hbm, ib0, ib1, ones_v, stage_v, acc_sh,
             sem_i0, sem_i1):
        c = lax.axis_index("c")
        s = lax.axis_index("s")
        wid = c * NS + s
        base = wid * CPW
        for j in range(CK // 16):
            ones_v[pl.ds(j * 16, 16)] = jnp.ones((16,), jnp.float32)
        for j in range(RPT // 16):
            stage_v[pl.ds(j * 16, 16)] = jnp.zeros((16,), jnp.float32)

        pltpu.sync_copy(stage_v, acc_sh.at[pl.ds(s * RPT, RPT)])
        plsc.subcore_barrier()

        pltpu.async_copy(ei_hbm.at[base, 1], ib0, sem_i0)

        @pl.loop(0, CPW, step=2)
        def _(k):
            cc = base + k
            pltpu.make_async_copy(ei_hbm.at[cc, 1], ib0, sem_i0).wait()
            pltpu.async_copy(ei_hbm.at[cc + 1, 1], ib1, sem_i1)
            pltpu.sync_copy(ones_v, acc_sh.at[ib0], add=True)
            pltpu.make_async_copy(ei_hbm.at[cc + 1, 1], ib1, sem_i1).wait()

            @pl.when(k + 2 < CPW)
            def _():
                pltpu.async_copy(ei_hbm.at[cc + 2, 1], ib0, sem_i0)

            pltpu.sync_copy(ones_v, acc_sh.at[ib1], add=True)

        plsc.subcore_barrier()
        pltpu.sync_copy(acc_sh.at[pl.ds(s * RPT, RPT)], stage_v)
        pltpu.sync_copy(stage_v, out_hbm.at[c, pl.ds(s * RPT, RPT)])

    return pl.kernel(
        body,
        out_type=jax.ShapeDtypeStruct((NC, NP), jnp.float32),
        mesh=_mesh(),
        scratch_types=[
            pltpu.VMEM((CK,), jnp.int32),
            pltpu.VMEM((CK,), jnp.int32),
            pltpu.VMEM((CK,), jnp.float32),
            pltpu.VMEM((RPT,), jnp.float32),
            pltpu.VMEM_SHARED((NP,), jnp.float32),
            pltpu.SemaphoreType.DMA,
            pltpu.SemaphoreType.DMA,
        ],
    )(ei3)


def _sc_edge_aggregate(ei3, table, dtype, vw):
    def body(ei_hbm, tbl_hbm, out_hbm,
             ib0, ib1, rows0, rows1, acc_sh,
             sem_i0, sem_i1, sem_g0, sem_g1, sem_s0, sem_s1, sem_w0, sem_w1):
        c = lax.axis_index("c")
        s = lax.axis_index("s")
        wid = c * NS + s
        base = wid * CPW

        @pl.loop(0, CK)
        def _(r):
            for j in range(IN // vw):
                rows0[r, pl.ds(j * vw, vw)] = jnp.zeros((vw,), dtype)

        for j in range(4):
            pltpu.sync_copy(rows0,
                            acc_sh.at[pl.ds(s * RPT_A + j * CK, CK), :])
        pltpu.sync_copy(rows0.at[pl.ds(0, RPT_A - 4 * CK), :],
                        acc_sh.at[pl.ds(s * RPT_A + 4 * CK, RPT_A - 4 * CK), :])
        plsc.subcore_barrier()

        pltpu.async_copy(ei_hbm.at[base], ib0, sem_i0)
        pltpu.make_async_copy(ei_hbm.at[base], ib0, sem_i0).wait()
        pltpu.async_copy(tbl_hbm.at[ib0.at[0]], rows0, sem_g0)
        pltpu.async_copy(ei_hbm.at[base + 1], ib1, sem_i1)

        @pl.loop(0, CPW, step=2)
        def _(k):
            cc = base + k
            pltpu.make_async_copy(ei_hbm.at[cc + 1], ib1, sem_i1).wait()
            pltpu.make_async_copy(tbl_hbm.at[ib0.at[0]], rows0, sem_g0).wait()
            pltpu.async_copy(tbl_hbm.at[ib1.at[0]], rows1, sem_g1)
            pltpu.async_copy(rows0, acc_sh.at[ib0.at[1]], sem_s0, add=True)
            pltpu.make_async_copy(rows0, acc_sh.at[ib0.at[1]], sem_s0).wait()

            @pl.when(k + 2 < CPW)
            def _():
                pltpu.async_copy(ei_hbm.at[cc + 2], ib0, sem_i0)

            pltpu.make_async_copy(tbl_hbm.at[ib1.at[0]], rows1, sem_g1).wait()
            pltpu.async_copy(rows1, acc_sh.at[ib1.at[1]], sem_s1, add=True)

            @pl.when(k + 2 < CPW)
            def _():
                pltpu.make_async_copy(ei_hbm.at[cc + 2], ib0, sem_i0).wait()
                pltpu.async_copy(tbl_hbm.at[ib0.at[0]], rows0, sem_g0)

            pltpu.make_async_copy(rows1, acc_sh.at[ib1.at[1]], sem_s1).wait()

            @pl.when(k + 2 < CPW)
            def _():
                pltpu.async_copy(ei_hbm.at[cc + 3], ib1, sem_i1)

        plsc.subcore_barrier()

        for j in range(5):
            nrows = CK if j < 4 else RPT_A - 4 * CK
            buf = rows0 if j % 2 == 0 else rows1
            sem = sem_w0 if j % 2 == 0 else sem_w1
            if j >= 2:
                pj = j - 2
                pnr = CK if pj < 4 else RPT_A - 4 * CK
                pbuf = rows0 if pj % 2 == 0 else rows1
                pltpu.make_async_copy(
                    pbuf.at[pl.ds(0, pnr), :],
                    out_hbm.at[c, pl.ds(s * RPT_A + pj * CK, pnr), :],
                    sem).wait()
            pltpu.sync_copy(acc_sh.at[pl.ds(s * RPT_A + j * CK, nrows), :],
                            buf.at[pl.ds(0, nrows), :])
            pltpu.async_copy(buf.at[pl.ds(0, nrows), :],
                             out_hbm.at[c, pl.ds(s * RPT_A + j * CK, nrows), :],
                             sem)
        for j in range(3, 5):
            nrows = CK if j < 4 else RPT_A - 4 * CK
            buf = rows0 if j % 2 == 0 else rows1
            sem = sem_w0 if j % 2 == 0 else sem_w1
            pltpu.make_async_copy(
                buf.at[pl.ds(0, nrows), :],
                out_hbm.at[c, pl.ds(s * RPT_A + j * CK, nrows), :],
                sem).wait()

    return pl.kernel(
        body,
        out_type=jax.ShapeDtypeStruct((NC, NP, IN), dtype),
        mesh=_mesh(),
        compiler_params=pltpu.CompilerParams(use_tc_tiling_on_sc=False),
        scratch_types=[
            pltpu.VMEM((2, CK), jnp.int32),
            pltpu.VMEM((2, CK), jnp.int32),
            pltpu.VMEM((CK, IN), dtype),
            pltpu.VMEM((CK, IN), dtype),
            pltpu.VMEM_SHARED((N_ACC, IN), dtype),
            pltpu.SemaphoreType.DMA,
            pltpu.SemaphoreType.DMA,
            pltpu.SemaphoreType.DMA,
            pltpu.SemaphoreType.DMA,
            pltpu.SemaphoreType.DMA,
            pltpu.SemaphoreType.DMA,
            pltpu.SemaphoreType.DMA,
            pltpu.SemaphoreType.DMA,
        ],
    )(ei3, table)


def _tc_prep(degp_t, xp):
    def body(dp_ref, x_ref, y_ref, dis_ref, dis2_ref):
        deg = dp_ref[:, 0:1] + dp_ref[:, 1:2]
        dis = jnp.where(deg > 0, lax.rsqrt(deg), 0.0)
        dis2 = lax.rsqrt(deg + 1.0)
        y_ref[...] = (dis * x_ref[...]).astype(T_AGG)
        dis_ref[...] = dis
        dis2_ref[...] = dis2

    grid = (NP // ROW_BLK,)
    return pl.pallas_call(
        body,
        grid=grid,
        in_specs=[
            pl.BlockSpec((ROW_BLK, NC), lambda i: (i, 0)),
            pl.BlockSpec((ROW_BLK, IN), lambda i: (i, 0)),
        ],
        out_specs=[
            pl.BlockSpec((ROW_BLK, IN), lambda i: (i, 0)),
            pl.BlockSpec((ROW_BLK, 1), lambda i: (i, 0)),
            pl.BlockSpec((ROW_BLK, 1), lambda i: (i, 0)),
        ],
        out_shape=[
            jax.ShapeDtypeStruct((NP, IN), T_AGG),
            jax.ShapeDtypeStruct((NP, 1), jnp.float32),
            jax.ShapeDtypeStruct((NP, 1), jnp.float32),
        ],
    )(degp_t, xp)


def _tc_dense(zp, dis, dis2, W1, b1r, W2):
    def body(zp_ref, dis_ref, dis2_ref, w1_ref, b1_ref, w2_ref, u_ref, ub_ref):
        zb = (zp_ref[0].astype(jnp.float32) + zp_ref[1].astype(jnp.float32))
        a = dis_ref[...] * zb
        h = lax.dot_general(a, w1_ref[...], (((1,), (1,)), ((), ())),
                            preferred_element_type=jnp.float32) + b1_ref[...]
        xw = lax.dot_general(h, w2_ref[...], (((1,), (1,)), ((), ())),
                             preferred_element_type=jnp.float32)
        u = dis2_ref[...] * xw
        u_ref[...] = u
        ub_ref[...] = u.astype(T_AGG)

    grid = (NP // ROW_BLK,)
    return pl.pallas_call(
        body,
        grid=grid,
        in_specs=[
            pl.BlockSpec((NC, ROW_BLK, IN), lambda i: (0, i, 0)),
            pl.BlockSpec((ROW_BLK, 1), lambda i: (i, 0)),
            pl.BlockSpec((ROW_BLK, 1), lambda i: (i, 0)),
            pl.BlockSpec((HID, IN), lambda i: (0, 0)),
            pl.BlockSpec((1, HID), lambda i: (0, 0)),
            pl.BlockSpec((OUT, HID), lambda i: (0, 0)),
        ],
        out_specs=[
            pl.BlockSpec((ROW_BLK, OUT), lambda i: (i, 0)),
            pl.BlockSpec((ROW_BLK, OUT), lambda i: (i, 0)),
        ],
        out_shape=[
            jax.ShapeDtypeStruct((NP, OUT), jnp.float32),
            jax.ShapeDtypeStruct((NP, OUT), T_AGG),
        ],
    )(zp, dis, dis2, W1, b1r, W2)


def _tc_final(vp, u, dis2, b2r):
    def body(vp_ref, u_ref, dis2_ref, b2_ref, out_ref):
        vb = (vp_ref[0].astype(jnp.float32) + vp_ref[1].astype(jnp.float32)
              + u_ref[...])
        out_ref[...] = dis2_ref[...] * vb + b2_ref[...]

    grid = (NP // ROW_BLK,)
    return pl.pallas_call(
        body,
        grid=grid,
        in_specs=[
            pl.BlockSpec((NC, ROW_BLK, OUT), lambda i: (0, i, 0)),
            pl.BlockSpec((ROW_BLK, OUT), lambda i: (i, 0)),
            pl.BlockSpec((ROW_BLK, 1), lambda i: (i, 0)),
            pl.BlockSpec((1, OUT), lambda i: (0, 0)),
        ],
        out_specs=pl.BlockSpec((ROW_BLK, OUT), lambda i: (i, 0)),
        out_shape=jax.ShapeDtypeStruct((NP, OUT), jnp.float32),
    )(vp, u, dis2, b2r)


def _edge_chunks(gidx, sidx):
    pad = E_PAD - E
    lanes = jnp.arange(pad, dtype=jnp.int32) % NTRASH
    g_all = jnp.concatenate([gidx, lanes])
    s_all = jnp.concatenate([sidx, N_ACC - NTRASH + lanes])
    return jnp.stack([g_all.reshape(NCH, CK), s_all.reshape(NCH, CK)], axis=1)


def kernel(x, edge_index, W1, b1, W2, b2):
    row = edge_index[0]
    col = edge_index[1]
    xp = jnp.pad(x, ((0, NP - N), (0, 0)))
    ei_z = _edge_chunks(col, row)
    ei_v = _edge_chunks(row, col)

    degp = _sc_histogram(ei_v)
    y, dis, dis2 = _tc_prep(jnp.transpose(degp), xp)
    zp = _sc_edge_aggregate(ei_z, y, T_AGG, VW_AGG)
    u, ub = _tc_dense(zp, dis, dis2, W1, b1.reshape(1, HID), W2)
    vp = _sc_edge_aggregate(ei_v, ub, T_AGG, VW_AGG)
    out = _tc_final(vp, u, dis2, b2.reshape(1, OUT))
    return out[:N]


# speedup vs baseline: 37.7280x; 1.0106x over previous
import jax
import jax.numpy as jnp
from jax import lax
from jax.experimental import pallas as pl
from jax.experimental.pallas import tpu as pltpu
from jax.experimental.pallas import tpu_sc as plsc

N = 10000
E = 320000
IN = 128
HID = 256
OUT = 128

NC, NS = 2, 16
NW = NC * NS
CK = 128
CPW = 80
NCH = NW * CPW
E_PAD = NCH * CK
RPT = 640
NP = NS * RPT
RPT_A = 632
N_ACC = NS * RPT_A
NTRASH = 64

NCHUNK_H = E // CK
BASE_H = NCHUNK_H // NW
EXTRA_H = NCHUNK_H - BASE_H * NW

ROW_BLK = 512

T_AGG = jnp.bfloat16
VW_AGG = 32


def _mesh():
    return plsc.VectorSubcoreMesh(
        core_axis_name="c", subcore_axis_name="s",
        num_cores=NC, num_subcores=NS)


def _sc_histogram(ei3):
    def body(ei_hbm, out_hbm, ib0, ib1, ones_v, stage_v, acc_sh,
             sem_i0, sem_i1):
        c = lax.axis_index("c")
        s = lax.axis_index("s")
        wid = c * NS + s
        base = wid * CPW
        for j in range(CK // 16):
            ones_v[pl.ds(j * 16, 16)] = jnp.ones((16,), jnp.float32)
        for j in range(RPT // 16):
            stage_v[pl.ds(j * 16, 16)] = jnp.zeros((16,), jnp.float32)

        pltpu.sync_copy(stage_v, acc_sh.at[pl.ds(s * RPT, RPT)])
        plsc.subcore_barrier()

        pltpu.async_copy(ei_hbm.at[base, 1], ib0, sem_i0)

        @pl.loop(0, CPW, step=2)
        def _(k):
            cc = base + k
            pltpu.make_async_copy(ei_hbm.at[cc, 1], ib0, sem_i0).wait()
            pltpu.async_copy(ei_hbm.at[cc + 1, 1], ib1, sem_i1)
            pltpu.sync_copy(ones_v, acc_sh.at[ib0], add=True)
            pltpu.make_async_copy(ei_hbm.at[cc + 1, 1], ib1, sem_i1).wait()

            @pl.when(k + 2 < CPW)
            def _():
                pltpu.async_copy(ei_hbm.at[cc + 2, 1], ib0, sem_i0)

            pltpu.sync_copy(ones_v, acc_sh.at[ib1], add=True)

        plsc.subcore_barrier()
        pltpu.sync_copy(acc_sh.at[pl.ds(s * RPT, RPT)], stage_v)
        pltpu.sync_copy(stage_v, out_hbm.at[c, pl.ds(s * RPT, RPT)])

    return pl.kernel(
        body,
        out_type=jax.ShapeDtypeStruct((NC, NP), jnp.float32),
        mesh=_mesh(),
        scratch_types=[
            pltpu.VMEM((CK,), jnp.int32),
            pltpu.VMEM((CK,), jnp.int32),
            pltpu.VMEM((CK,), jnp.float32),
            pltpu.VMEM((RPT,), jnp.float32),
            pltpu.VMEM_SHARED((NP,), jnp.float32),
            pltpu.SemaphoreType.DMA,
            pltpu.SemaphoreType.DMA,
        ],
    )(ei3)


def _sc_edge_aggregate(ei3, table, dtype, vw):
    def body(ei_hbm, tbl_hbm, out_hbm,
             ib0, ib1, rows0, rows1, acc_sh,
             sem_i0, sem_i1, sem_g0, sem_g1, sem_s0, sem_s1, sem_w0, sem_w1):
        c = lax.axis_index("c")
        s = lax.axis_index("s")
        wid = c * NS + s
        base = wid * CPW

        @pl.loop(0, CK)
        def _(r):
            for j in range(IN // vw):
                rows0[r, pl.ds(j * vw, vw)] = jnp.zeros((vw,), dtype)

        for j in range(4):
            pltpu.sync_copy(rows0,
                            acc_sh.at[pl.ds(s * RPT_A + j * CK, CK), :])
        pltpu.sync_copy(rows0.at[pl.ds(0, RPT_A - 4 * CK), :],
                        acc_sh.at[pl.ds(s * RPT_A + 4 * CK, RPT_A - 4 * CK), :])
        plsc.subcore_barrier()

        pltpu.async_copy(ei_hbm.at[base], ib0, sem_i0)
        pltpu.make_async_copy(ei_hbm.at[base], ib0, sem_i0).wait()
        pltpu.async_copy(tbl_hbm.at[ib0.at[0]], rows0, sem_g0)
        pltpu.async_copy(ei_hbm.at[base + 1], ib1, sem_i1)

        @pl.loop(0, CPW, step=2)
        def _(k):
            cc = base + k
            pltpu.make_async_copy(ei_hbm.at[cc + 1], ib1, sem_i1).wait()
            pltpu.make_async_copy(tbl_hbm.at[ib0.at[0]], rows0, sem_g0).wait()
            pltpu.async_copy(tbl_hbm.at[ib1.at[0]], rows1, sem_g1)
            pltpu.async_copy(rows0, acc_sh.at[ib0.at[1]], sem_s0, add=True)
            pltpu.make_async_copy(rows0, acc_sh.at[ib0.at[1]], sem_s0).wait()

            @pl.when(k + 2 < CPW)
            def _():
                pltpu.async_copy(ei_hbm.at[cc + 2], ib0, sem_i0)

            pltpu.make_async_copy(tbl_hbm.at[ib1.at[0]], rows1, sem_g1).wait()
            pltpu.async_copy(rows1, acc_sh.at[ib1.at[1]], sem_s1, add=True)

            @pl.when(k + 2 < CPW)
            def _():
                pltpu.make_async_copy(ei_hbm.at[cc + 2], ib0, sem_i0).wait()
                pltpu.async_copy(tbl_hbm.at[ib0.at[0]], rows0, sem_g0)

            pltpu.make_async_copy(rows1, acc_sh.at[ib1.at[1]], sem_s1).wait()

            @pl.when(k + 2 < CPW)
            def _():
                pltpu.async_copy(ei_hbm.at[cc + 3], ib1, sem_i1)

        plsc.subcore_barrier()

        for j in range(5):
            nrows = CK if j < 4 else RPT_A - 4 * CK
            buf = rows0 if j % 2 == 0 else rows1
            sem = sem_w0 if j % 2 == 0 else sem_w1
            if j >= 2:
                pj = j - 2
                pnr = CK if pj < 4 else RPT_A - 4 * CK
                pbuf = rows0 if pj % 2 == 0 else rows1
                pltpu.make_async_copy(
                    pbuf.at[pl.ds(0, pnr), :],
                    out_hbm.at[c, pl.ds(s * RPT_A + pj * CK, pnr), :],
                    sem).wait()
            pltpu.sync_copy(acc_sh.at[pl.ds(s * RPT_A + j * CK, nrows), :],
                            buf.at[pl.ds(0, nrows), :])
            pltpu.async_copy(buf.at[pl.ds(0, nrows), :],
                             out_hbm.at[c, pl.ds(s * RPT_A + j * CK, nrows), :],
                             sem)
        for j in range(3, 5):
            nrows = CK if j < 4 else RPT_A - 4 * CK
            buf = rows0 if j % 2 == 0 else rows1
            sem = sem_w0 if j % 2 == 0 else sem_w1
            pltpu.make_async_copy(
                buf.at[pl.ds(0, nrows), :],
                out_hbm.at[c, pl.ds(s * RPT_A + j * CK, nrows), :],
                sem).wait()

    return pl.kernel(
        body,
        out_type=jax.ShapeDtypeStruct((NC, NP, IN), dtype),
        mesh=_mesh(),
        compiler_params=pltpu.CompilerParams(use_tc_tiling_on_sc=False),
        scratch_types=[
            pltpu.VMEM((2, CK), jnp.int32),
            pltpu.VMEM((2, CK), jnp.int32),
            pltpu.VMEM((CK, IN), dtype),
            pltpu.VMEM((CK, IN), dtype),
            pltpu.VMEM_SHARED((N_ACC, IN), dtype),
            pltpu.SemaphoreType.DMA,
            pltpu.SemaphoreType.DMA,
            pltpu.SemaphoreType.DMA,
            pltpu.SemaphoreType.DMA,
            pltpu.SemaphoreType.DMA,
            pltpu.SemaphoreType.DMA,
            pltpu.SemaphoreType.DMA,
            pltpu.SemaphoreType.DMA,
        ],
    )(ei3, table)


def _tc_prep(degp_t, xp):
    def body(dp_ref, x_ref, y_ref, dis_ref, dis2_ref):
        deg = dp_ref[:, 0:1] + dp_ref[:, 1:2]
        dis = jnp.where(deg > 0, lax.rsqrt(deg), 0.0)
        dis2 = lax.rsqrt(deg + 1.0)
        y_ref[...] = (dis * x_ref[...]).astype(T_AGG)
        dis_ref[...] = dis
        dis2_ref[...] = dis2

    grid = (NP // ROW_BLK,)
    return pl.pallas_call(
        body,
        grid=grid,
        in_specs=[
            pl.BlockSpec((ROW_BLK, NC), lambda i: (i, 0)),
            pl.BlockSpec((ROW_BLK, IN), lambda i: (i, 0)),
        ],
        out_specs=[
            pl.BlockSpec((ROW_BLK, IN), lambda i: (i, 0)),
            pl.BlockSpec((ROW_BLK, 1), lambda i: (i, 0)),
            pl.BlockSpec((ROW_BLK, 1), lambda i: (i, 0)),
        ],
        out_shape=[
            jax.ShapeDtypeStruct((NP, IN), T_AGG),
            jax.ShapeDtypeStruct((NP, 1), jnp.float32),
            jax.ShapeDtypeStruct((NP, 1), jnp.float32),
        ],
    )(degp_t, xp)


def _tc_dense(zp, dis, dis2, W1, b1r, W2):
    def body(zp_ref, dis_ref, dis2_ref, w1_ref, b1_ref, w2_ref, u_ref, ub_ref):
        zb = (zp_ref[0].astype(jnp.float32) + zp_ref[1].astype(jnp.float32))
        a = dis_ref[...] * zb
        h = lax.dot_general(a, w1_ref[...], (((1,), (1,)), ((), ())),
                            preferred_element_type=jnp.float32) + b1_ref[...]
        xw = lax.dot_general(h, w2_ref[...], (((1,), (1,)), ((), ())),
                             preferred_element_type=jnp.float32)
        u = dis2_ref[...] * xw
        u_ref[...] = u
        ub_ref[...] = u.astype(T_AGG)

    grid = (NP // ROW_BLK,)
    return pl.pallas_call(
        body,
        grid=grid,
        in_specs=[
            pl.BlockSpec((NC, ROW_BLK, IN), lambda i: (0, i, 0)),
            pl.BlockSpec((ROW_BLK, 1), lambda i: (i, 0)),
            pl.BlockSpec((ROW_BLK, 1), lambda i: (i, 0)),
            pl.BlockSpec((HID, IN), lambda i: (0, 0)),
            pl.BlockSpec((1, HID), lambda i: (0, 0)),
            pl.BlockSpec((OUT, HID), lambda i: (0, 0)),
        ],
        out_specs=[
            pl.BlockSpec((ROW_BLK, OUT), lambda i: (i, 0)),
            pl.BlockSpec((ROW_BLK, OUT), lambda i: (i, 0)),
        ],
        out_shape=[
            jax.ShapeDtypeStruct((NP, OUT), jnp.float32),
            jax.ShapeDtypeStruct((NP, OUT), T_AGG),
        ],
    )(zp, dis, dis2, W1, b1r, W2)


def _tc_final(vp, u, dis2, b2r):
    def body(vp_ref, u_ref, dis2_ref, b2_ref, out_ref):
        vb = (vp_ref[0].astype(jnp.float32) + vp_ref[1].astype(jnp.float32)
              + u_ref[...])
        out_ref[...] = dis2_ref[...] * vb + b2_ref[...]

    grid = (NP // ROW_BLK,)
    return pl.pallas_call(
        body,
        grid=grid,
        in_specs=[
            pl.BlockSpec((NC, ROW_BLK, OUT), lambda i: (0, i, 0)),
            pl.BlockSpec((ROW_BLK, OUT), lambda i: (i, 0)),
            pl.BlockSpec((ROW_BLK, 1), lambda i: (i, 0)),
            pl.BlockSpec((1, OUT), lambda i: (0, 0)),
        ],
        out_specs=pl.BlockSpec((ROW_BLK, OUT), lambda i: (i, 0)),
        out_shape=jax.ShapeDtypeStruct((N, OUT), jnp.float32),
    )(vp, u, dis2, b2r)


def _edge_chunks(gidx, sidx):
    pad = E_PAD - E
    lanes = jnp.arange(pad, dtype=jnp.int32) % NTRASH
    g_all = jnp.concatenate([gidx, lanes])
    s_all = jnp.concatenate([sidx, N_ACC - NTRASH + lanes])
    return jnp.stack([g_all.reshape(NCH, CK), s_all.reshape(NCH, CK)], axis=1)


def kernel(x, edge_index, W1, b1, W2, b2):
    row = edge_index[0]
    col = edge_index[1]
    xp = jnp.pad(x, ((0, NP - N), (0, 0)))
    ei_z = _edge_chunks(col, row)
    ei_v = _edge_chunks(row, col)

    degp = _sc_histogram(ei_v)
    y, dis, dis2 = _tc_prep(jnp.transpose(degp), xp)
    zp = _sc_edge_aggregate(ei_z, y, T_AGG, VW_AGG)
    u, ub = _tc_dense(zp, dis, dis2, W1, b1.reshape(1, HID), W2)
    vp = _sc_edge_aggregate(ei_v, ub, T_AGG, VW_AGG)
    return _tc_final(vp, u, dis2, b2.reshape(1, OUT))
